# Initial kernel scaffold; baseline (speedup 1.0000x reference)
import jax
import jax.numpy as jnp
from jax.experimental import pallas as pl


def kernel(x, edge_index, W1, b1, W2, b2):
    raise NotImplementedError("write your pallas kernel here")


# speedup vs baseline: 7.2514x; 7.2514x over previous
import jax
import jax.numpy as jnp
from jax import lax
from jax.experimental import pallas as pl
from jax.experimental.pallas import tpu as pltpu
from jax.experimental.pallas import tpu_sc as plsc

N = 10000
E = 320000
D_IN = 128
D_H = 128
D_OUT = 16

NC = 2
NS = 16
NW = NC * NS
EPW = E // NW
K = 80
NCH = EPW // K
NP = 10240
RPT = NP // NS

_MESH = dict(core_axis_name="c", subcore_axis_name="s",
             num_cores=NC, num_subcores=NS)


def _deg_body(src_hbm, dst_hbm, zeros_hbm, degs_out, degd_out,
              idx_v, ones_v, degs_sh, degd_sh):
    cid = lax.axis_index("c")
    sid = lax.axis_index("s")
    wid = cid * NS + sid
    pltpu.sync_copy(zeros_hbm, degs_sh.at[pl.ds(sid * RPT, RPT)])
    pltpu.sync_copy(zeros_hbm, degd_sh.at[pl.ds(sid * RPT, RPT)])

    def _fill(i, c):
        ones_v[pl.ds(i * 16, 16)] = jnp.ones((16,), jnp.float32)
        return c
    lax.fori_loop(0, K // 16, _fill, 0)
    plsc.subcore_barrier()

    pltpu.sync_copy(src_hbm.at[wid], idx_v)

    def _s(j, c):
        pltpu.sync_copy(ones_v, degs_sh.at[idx_v.at[j]], add=True)
        return c
    lax.fori_loop(0, NCH, _s, 0)

    pltpu.sync_copy(dst_hbm.at[wid], idx_v)

    def _d(j, c):
        pltpu.sync_copy(ones_v, degd_sh.at[idx_v.at[j]], add=True)
        return c
    lax.fori_loop(0, NCH, _d, 0)

    plsc.subcore_barrier()
    pltpu.sync_copy(degs_sh.at[pl.ds(sid * RPT, RPT)],
                    degs_out.at[cid, pl.ds(sid * RPT, RPT)])
    pltpu.sync_copy(degd_sh.at[pl.ds(sid * RPT, RPT)],
                    degd_out.at[cid, pl.ds(sid * RPT, RPT)])


def _make_agg_body(d):
    def _agg_body(x_hbm, src_hbm, dst_hbm, zeros_hbm, out_hbm,
                  sidx_v, didx_v, rows_v, acc_sh, sem):
        cid = lax.axis_index("c")
        sid = lax.axis_index("s")
        wid = cid * NS + sid
        pltpu.sync_copy(zeros_hbm, acc_sh.at[pl.ds(sid * RPT, RPT), :])
        pltpu.sync_copy(src_hbm.at[wid], sidx_v)
        pltpu.sync_copy(dst_hbm.at[wid], didx_v)
        plsc.subcore_barrier()

        def _j(j, c):
            pltpu.async_copy(x_hbm.at[sidx_v.at[j]], rows_v, sem).wait()
            pltpu.sync_copy(rows_v, acc_sh.at[didx_v.at[j]], add=True)
            return c
        lax.fori_loop(0, NCH, _j, 0)

        plsc.subcore_barrier()
        pltpu.sync_copy(acc_sh.at[pl.ds(sid * RPT, RPT), :],
                        out_hbm.at[cid, pl.ds(sid * RPT, RPT), :])
    return _agg_body


def _prep_body(x_ref, ds_ref, dd_ref, xs_ref, ns_ref, nd_ref):
    ns = lax.rsqrt(jnp.maximum(ds_ref[0] + ds_ref[1], 1.0))
    nd = lax.rsqrt(jnp.maximum(dd_ref[0] + dd_ref[1], 1.0))
    xs_ref[...] = x_ref[...] * ns
    ns_ref[...] = ns
    nd_ref[...] = nd


def _mlp_body(a0_ref, nd_ref, ns_ref, w1_ref, b1_ref, w2_ref, out_ref):
    a = (a0_ref[0] + a0_ref[1]) * nd_ref[...]
    h = jnp.dot(a, w1_ref[...], preferred_element_type=jnp.float32,
                precision=lax.Precision.HIGHEST)
    h = jnp.maximum(h + b1_ref[...], 0.0)
    out_ref[...] = jnp.dot(h * ns_ref[...], w2_ref[...],
                           preferred_element_type=jnp.float32,
                           precision=lax.Precision.HIGHEST)


def _final_body(a_ref, nd_ref, b2_ref, out_ref):
    out_ref[...] = (a_ref[0] + a_ref[1]) * nd_ref[...] + b2_ref[...]


def kernel(x, edge_index, W1, b1, W2, b2):
    src = edge_index[0].reshape(NW, NCH, K)
    dst = edge_index[1].reshape(NW, NCH, K)
    zeros_r = jnp.zeros((RPT,), jnp.float32)
    zeros_h = jnp.zeros((RPT, D_H), jnp.float32)
    zeros_o = jnp.zeros((RPT, D_OUT), jnp.float32)

    mesh = plsc.VectorSubcoreMesh(**_MESH)
    degs_p, degd_p = pl.kernel(
        _deg_body,
        out_type=(jax.ShapeDtypeStruct((NC, NP), jnp.float32),
                  jax.ShapeDtypeStruct((NC, NP), jnp.float32)),
        mesh=mesh,
        scratch_types=(
            pltpu.VMEM((NCH, K), jnp.int32),
            pltpu.VMEM((K,), jnp.float32),
            pltpu.VMEM_SHARED((NP,), jnp.float32),
            pltpu.VMEM_SHARED((NP,), jnp.float32),
        ),
    )(src, dst, zeros_r)

    B = 400
    G = N // B
    degs3 = degs_p.reshape(NC, NP, 1)
    degd3 = degd_p.reshape(NC, NP, 1)
    xs, ns, nd = pl.pallas_call(
        _prep_body,
        grid=(G,),
        in_specs=[pl.BlockSpec((B, D_IN), lambda i: (i, 0)),
                  pl.BlockSpec((NC, B, 1), lambda i: (0, i, 0)),
                  pl.BlockSpec((NC, B, 1), lambda i: (0, i, 0))],
        out_specs=[pl.BlockSpec((B, D_IN), lambda i: (i, 0)),
                   pl.BlockSpec((B, 1), lambda i: (i, 0)),
                   pl.BlockSpec((B, 1), lambda i: (i, 0))],
        out_shape=[jax.ShapeDtypeStruct((N, D_IN), jnp.float32),
                   jax.ShapeDtypeStruct((N, 1), jnp.float32),
                   jax.ShapeDtypeStruct((N, 1), jnp.float32)],
    )(x, degs3, degd3)

    mesh = plsc.VectorSubcoreMesh(**_MESH)
    agg1 = pl.kernel(
        _make_agg_body(D_H),
        out_type=jax.ShapeDtypeStruct((NC, NP, D_H), jnp.float32),
        mesh=mesh,
        scratch_types=(
            pltpu.VMEM((NCH, K), jnp.int32),
            pltpu.VMEM((NCH, K), jnp.int32),
            pltpu.VMEM((K, D_H), jnp.float32),
            pltpu.VMEM_SHARED((NP, D_H), jnp.float32),
            pltpu.SemaphoreType.DMA,
        ),
    )(xs, src, dst, zeros_h)

    b1r = b1.reshape(1, D_H)
    y2 = pl.pallas_call(
        _mlp_body,
        grid=(G,),
        in_specs=[pl.BlockSpec((NC, B, D_H), lambda i: (0, i, 0)),
                  pl.BlockSpec((B, 1), lambda i: (i, 0)),
                  pl.BlockSpec((B, 1), lambda i: (i, 0)),
                  pl.BlockSpec((D_H, D_H), lambda i: (0, 0)),
                  pl.BlockSpec((1, D_H), lambda i: (0, 0)),
                  pl.BlockSpec((D_H, D_OUT), lambda i: (0, 0))],
        out_specs=pl.BlockSpec((B, D_OUT), lambda i: (i, 0)),
        out_shape=jax.ShapeDtypeStruct((N, D_OUT), jnp.float32),
    )(agg1, nd, ns, W1, b1r, W2)

    mesh = plsc.VectorSubcoreMesh(**_MESH)
    agg2 = pl.kernel(
        _make_agg_body(D_OUT),
        out_type=jax.ShapeDtypeStruct((NC, NP, D_OUT), jnp.float32),
        mesh=mesh,
        compiler_params=pltpu.CompilerParams(use_tc_tiling_on_sc=False),
        scratch_types=(
            pltpu.VMEM((NCH, K), jnp.int32),
            pltpu.VMEM((NCH, K), jnp.int32),
            pltpu.VMEM((K, D_OUT), jnp.float32),
            pltpu.VMEM_SHARED((NP, D_OUT), jnp.float32),
            pltpu.SemaphoreType.DMA,
        ),
    )(y2, src, dst, zeros_o)

    b2r = b2.reshape(1, D_OUT)
    out = pl.pallas_call(
        _final_body,
        grid=(G,),
        in_specs=[pl.BlockSpec((NC, B, D_OUT), lambda i: (0, i, 0)),
                  pl.BlockSpec((B, 1), lambda i: (i, 0)),
                  pl.BlockSpec((1, D_OUT), lambda i: (0, 0))],
        out_specs=pl.BlockSpec((B, D_OUT), lambda i: (i, 0)),
        out_shape=jax.ShapeDtypeStruct((N, D_OUT), jnp.float32),
    )(agg2, nd, b2r)
    return out


# speedup vs baseline: 11.5167x; 1.5882x over previous
import jax
import jax.numpy as jnp
from jax import lax
from jax.experimental import pallas as pl
from jax.experimental.pallas import tpu as pltpu
from jax.experimental.pallas import tpu_sc as plsc

N = 10000
E = 320000
D_IN = 128
D_H = 128
D_OUT = 16

NC = 2
NS = 16
NW = NC * NS
EPW = E // NW
K = 80
NCH = EPW // K
NP = 10240
RPT = NP // NS

_MESH = dict(core_axis_name="c", subcore_axis_name="s",
             num_cores=NC, num_subcores=NS)


NBUF = 5


def _deg_body(src_hbm, dst_hbm, zeros_hbm, degs_out, degd_out,
              idx_v, ones_v, degs_sh, degd_sh, *sems):
    cid = lax.axis_index("c")
    sid = lax.axis_index("s")
    wid = cid * NS + sid
    pltpu.sync_copy(zeros_hbm, degs_sh.at[pl.ds(sid * RPT, RPT)])
    pltpu.sync_copy(zeros_hbm, degd_sh.at[pl.ds(sid * RPT, RPT)])

    def _fill(i, c):
        ones_v[pl.ds(i * 16, 16)] = jnp.ones((16,), jnp.float32)
        return c
    lax.fori_loop(0, K // 16, _fill, 0)
    plsc.subcore_barrier()

    pltpu.sync_copy(src_hbm.at[wid], idx_v)

    def _scatter_ones(deg_sh):
        def _g(g, c):
            for b in range(NBUF):
                j = g * NBUF + b
                pltpu.async_copy(ones_v, deg_sh.at[idx_v.at[j]], sems[b],
                                 add=True)
            for b in range(NBUF):
                j = g * NBUF + b
                pltpu.make_async_copy(ones_v, deg_sh.at[idx_v.at[j]],
                                      sems[b]).wait()
            return c
        lax.fori_loop(0, NCH // NBUF, _g, 0)

    _scatter_ones(degs_sh)
    pltpu.sync_copy(dst_hbm.at[wid], idx_v)
    _scatter_ones(degd_sh)

    plsc.subcore_barrier()
    pltpu.sync_copy(degs_sh.at[pl.ds(sid * RPT, RPT)],
                    degs_out.at[cid, pl.ds(sid * RPT, RPT)])
    pltpu.sync_copy(degd_sh.at[pl.ds(sid * RPT, RPT)],
                    degd_out.at[cid, pl.ds(sid * RPT, RPT)])


def _agg1_body(x_hbm, src_hbm, dst_hbm, zeros_hbm, out_hbm,
               sidx_v, didx_v, rows_v, acc_sh,
               gsem0, gsem1, dsem0, dsem1):
    gsems = (gsem0, gsem1)
    dsems = (dsem0, dsem1)
    cid = lax.axis_index("c")
    sid = lax.axis_index("s")
    wid = cid * NS + sid
    pltpu.sync_copy(zeros_hbm, acc_sh.at[pl.ds(sid * RPT, RPT), :])
    pltpu.sync_copy(src_hbm.at[wid], sidx_v)
    plsc.subcore_barrier()

    for b in range(2):
        pltpu.async_copy(dst_hbm.at[wid, pl.ds(b, 1)], didx_v.at[b], dsems[b])
        pltpu.async_copy(x_hbm.at[sidx_v.at[b]], rows_v.at[b], gsems[b])

    def _step(j, b):
        pltpu.make_async_copy(x_hbm.at[sidx_v.at[j]], rows_v.at[b],
                              gsems[b]).wait()
        pltpu.make_async_copy(dst_hbm.at[wid, pl.ds(j, 1)], didx_v.at[b],
                              dsems[b]).wait()
        pltpu.sync_copy(rows_v.at[b], acc_sh.at[didx_v.at[b, 0]], add=True)

        @pl.when(j + 2 < NCH)
        def _():
            pltpu.async_copy(dst_hbm.at[wid, pl.ds(j + 2, 1)], didx_v.at[b],
                             dsems[b])
            pltpu.async_copy(x_hbm.at[sidx_v.at[j + 2]], rows_v.at[b],
                             gsems[b])

    def _g(g, c):
        for b in range(2):
            _step(g * 2 + b, b)
        return c
    lax.fori_loop(0, NCH // 2, _g, 0)
    _step(NCH - 1, (NCH - 1) % 2)

    plsc.subcore_barrier()
    pltpu.sync_copy(acc_sh.at[pl.ds(sid * RPT, RPT), :],
                    out_hbm.at[cid, pl.ds(sid * RPT, RPT), :])


def _make_agg_body(d):
    def _agg_body(x_hbm, src_hbm, dst_hbm, zeros_hbm, out_hbm,
                  sidx_v, didx_v, rows_v, acc_sh, *sems):
        cid = lax.axis_index("c")
        sid = lax.axis_index("s")
        wid = cid * NS + sid
        pltpu.sync_copy(zeros_hbm, acc_sh.at[pl.ds(sid * RPT, RPT), :])
        pltpu.sync_copy(src_hbm.at[wid], sidx_v)
        pltpu.sync_copy(dst_hbm.at[wid], didx_v)
        plsc.subcore_barrier()

        for b in range(NBUF):
            pltpu.async_copy(x_hbm.at[sidx_v.at[b]], rows_v.at[b], sems[b])

        def _g(g, c):
            for b in range(NBUF):
                j = g * NBUF + b
                pltpu.make_async_copy(x_hbm.at[sidx_v.at[j]], rows_v.at[b],
                                      sems[b]).wait()
                pltpu.sync_copy(rows_v.at[b], acc_sh.at[didx_v.at[j]],
                                add=True)

                @pl.when(j + NBUF < NCH)
                def _():
                    pltpu.async_copy(x_hbm.at[sidx_v.at[j + NBUF]],
                                     rows_v.at[b], sems[b])
            return c
        lax.fori_loop(0, NCH // NBUF, _g, 0)

        plsc.subcore_barrier()
        pltpu.sync_copy(acc_sh.at[pl.ds(sid * RPT, RPT), :],
                        out_hbm.at[cid, pl.ds(sid * RPT, RPT), :])
    return _agg_body


def _prep_body(x_ref, ds_ref, dd_ref, xs_ref, ns_ref, nd_ref):
    ns = lax.rsqrt(jnp.maximum(ds_ref[0] + ds_ref[1], 1.0))
    nd = lax.rsqrt(jnp.maximum(dd_ref[0] + dd_ref[1], 1.0))
    xs_ref[...] = x_ref[...] * ns
    ns_ref[...] = ns
    nd_ref[...] = nd


def _mlp_body(a0_ref, nd_ref, ns_ref, w1_ref, b1_ref, w2_ref, out_ref):
    a = (a0_ref[0] + a0_ref[1]) * nd_ref[...]
    h = jnp.dot(a, w1_ref[...], preferred_element_type=jnp.float32,
                precision=lax.Precision.HIGHEST)
    h = jnp.maximum(h + b1_ref[...], 0.0)
    out_ref[...] = jnp.dot(h * ns_ref[...], w2_ref[...],
                           preferred_element_type=jnp.float32,
                           precision=lax.Precision.HIGHEST)


def _final_body(a_ref, nd_ref, b2_ref, out_ref):
    out_ref[...] = (a_ref[0] + a_ref[1]) * nd_ref[...] + b2_ref[...]


def kernel(x, edge_index, W1, b1, W2, b2):
    src = edge_index[0].reshape(NW, NCH, K)
    dst = edge_index[1].reshape(NW, NCH, K)
    zeros_r = jnp.zeros((RPT,), jnp.float32)
    zeros_h = jnp.zeros((RPT, D_H), jnp.float32)
    zeros_o = jnp.zeros((RPT, D_OUT), jnp.float32)

    mesh = plsc.VectorSubcoreMesh(**_MESH)
    degs_p, degd_p = pl.kernel(
        _deg_body,
        out_type=(jax.ShapeDtypeStruct((NC, NP), jnp.float32),
                  jax.ShapeDtypeStruct((NC, NP), jnp.float32)),
        mesh=mesh,
        scratch_types=(
            pltpu.VMEM((NCH, K), jnp.int32),
            pltpu.VMEM((K,), jnp.float32),
            pltpu.VMEM_SHARED((NP,), jnp.float32),
            pltpu.VMEM_SHARED((NP,), jnp.float32),
        ) + (pltpu.SemaphoreType.DMA,) * NBUF,
    )(src, dst, zeros_r)

    B = 400
    G = N // B
    degs3 = degs_p.reshape(NC, NP, 1)
    degd3 = degd_p.reshape(NC, NP, 1)
    xs, ns, nd = pl.pallas_call(
        _prep_body,
        grid=(G,),
        in_specs=[pl.BlockSpec((B, D_IN), lambda i: (i, 0)),
                  pl.BlockSpec((NC, B, 1), lambda i: (0, i, 0)),
                  pl.BlockSpec((NC, B, 1), lambda i: (0, i, 0))],
        out_specs=[pl.BlockSpec((B, D_IN), lambda i: (i, 0)),
                   pl.BlockSpec((B, 1), lambda i: (i, 0)),
                   pl.BlockSpec((B, 1), lambda i: (i, 0))],
        out_shape=[jax.ShapeDtypeStruct((N, D_IN), jnp.float32),
                   jax.ShapeDtypeStruct((N, 1), jnp.float32),
                   jax.ShapeDtypeStruct((N, 1), jnp.float32)],
    )(x, degs3, degd3)

    mesh = plsc.VectorSubcoreMesh(**_MESH)
    agg1 = pl.kernel(
        _agg1_body,
        out_type=jax.ShapeDtypeStruct((NC, NP, D_H), jnp.float32),
        mesh=mesh,
        scratch_types=(
            pltpu.VMEM((NCH, K), jnp.int32),
            pltpu.VMEM((2, 1, K), jnp.int32),
            pltpu.VMEM((2, K, D_H), jnp.float32),
            pltpu.VMEM_SHARED((NP, D_H), jnp.float32),
        ) + (pltpu.SemaphoreType.DMA,) * 4,
    )(xs, src, dst, zeros_h)

    b1r = b1.reshape(1, D_H)
    y2 = pl.pallas_call(
        _mlp_body,
        grid=(G,),
        in_specs=[pl.BlockSpec((NC, B, D_H), lambda i: (0, i, 0)),
                  pl.BlockSpec((B, 1), lambda i: (i, 0)),
                  pl.BlockSpec((B, 1), lambda i: (i, 0)),
                  pl.BlockSpec((D_H, D_H), lambda i: (0, 0)),
                  pl.BlockSpec((1, D_H), lambda i: (0, 0)),
                  pl.BlockSpec((D_H, D_OUT), lambda i: (0, 0))],
        out_specs=pl.BlockSpec((B, D_OUT), lambda i: (i, 0)),
        out_shape=jax.ShapeDtypeStruct((N, D_OUT), jnp.float32),
    )(agg1, nd, ns, W1, b1r, W2)

    mesh = plsc.VectorSubcoreMesh(**_MESH)
    agg2 = pl.kernel(
        _make_agg_body(D_OUT),
        out_type=jax.ShapeDtypeStruct((NC, NP, D_OUT), jnp.float32),
        mesh=mesh,
        compiler_params=pltpu.CompilerParams(use_tc_tiling_on_sc=False),
        scratch_types=(
            pltpu.VMEM((NCH, K), jnp.int32),
            pltpu.VMEM((NCH, K), jnp.int32),
            pltpu.VMEM((NBUF, K, D_OUT), jnp.float32),
            pltpu.VMEM_SHARED((NP, D_OUT), jnp.float32),
        ) + (pltpu.SemaphoreType.DMA,) * NBUF,
    )(y2, src, dst, zeros_o)

    b2r = b2.reshape(1, D_OUT)
    out = pl.pallas_call(
        _final_body,
        grid=(G,),
        in_specs=[pl.BlockSpec((NC, B, D_OUT), lambda i: (0, i, 0)),
                  pl.BlockSpec((B, 1), lambda i: (i, 0)),
                  pl.BlockSpec((1, D_OUT), lambda i: (0, 0))],
        out_specs=pl.BlockSpec((B, D_OUT), lambda i: (i, 0)),
        out_shape=jax.ShapeDtypeStruct((N, D_OUT), jnp.float32),
    )(agg2, nd, b2r)
    return out


# speedup vs baseline: 13.0730x; 1.1351x over previous
import jax
import jax.numpy as jnp
from jax import lax
from jax.experimental import pallas as pl
from jax.experimental.pallas import tpu as pltpu
from jax.experimental.pallas import tpu_sc as plsc

N = 10000
E = 320000
D_IN = 128
D_H = 128
D_OUT = 16

NC = 2
NS = 16
NW = NC * NS
EPW = E // NW
K = 80
NCH = EPW // K
NP = 10240
RPT = NP // NS

_MESH = dict(core_axis_name="c", subcore_axis_name="s",
             num_cores=NC, num_subcores=NS)


NBUF = 5


def _deg_body(src_hbm, dst_hbm, zeros_hbm, degs_out, degd_out,
              idx_v, ones_v, degs_sh, degd_sh, *sems):
    cid = lax.axis_index("c")
    sid = lax.axis_index("s")
    wid = cid * NS + sid
    pltpu.sync_copy(zeros_hbm, degs_sh.at[pl.ds(sid * RPT, RPT)])
    pltpu.sync_copy(zeros_hbm, degd_sh.at[pl.ds(sid * RPT, RPT)])

    def _fill(i, c):
        ones_v[pl.ds(i * 16, 16)] = jnp.ones((16,), jnp.float32)
        return c
    lax.fori_loop(0, K // 16, _fill, 0)
    plsc.subcore_barrier()

    pltpu.sync_copy(src_hbm.at[wid], idx_v)

    def _scatter_ones(deg_sh):
        def _g(g, c):
            for b in range(NBUF):
                j = g * NBUF + b
                pltpu.async_copy(ones_v, deg_sh.at[idx_v.at[j]], sems[b],
                                 add=True)
            for b in range(NBUF):
                j = g * NBUF + b
                pltpu.make_async_copy(ones_v, deg_sh.at[idx_v.at[j]],
                                      sems[b]).wait()
            return c
        lax.fori_loop(0, NCH // NBUF, _g, 0)

    _scatter_ones(degs_sh)
    pltpu.sync_copy(dst_hbm.at[wid], idx_v)
    _scatter_ones(degd_sh)

    plsc.subcore_barrier()
    pltpu.sync_copy(degs_sh.at[pl.ds(sid * RPT, RPT)],
                    degs_out.at[cid, pl.ds(sid * RPT, RPT)])
    pltpu.sync_copy(degd_sh.at[pl.ds(sid * RPT, RPT)],
                    degd_out.at[cid, pl.ds(sid * RPT, RPT)])


def _agg1_body(x_hbm, src_hbm, dst_hbm, zeros_hbm, out_hbm,
               sidx_v, didx_v, rows_v, acc_sh,
               gsem0, gsem1, dsem0, dsem1):
    gsems = (gsem0, gsem1)
    dsems = (dsem0, dsem1)
    cid = lax.axis_index("c")
    sid = lax.axis_index("s")
    wid = cid * NS + sid
    pltpu.sync_copy(zeros_hbm, acc_sh.at[pl.ds(sid * RPT, RPT), :])
    pltpu.sync_copy(src_hbm.at[wid], sidx_v)
    plsc.subcore_barrier()

    for b in range(2):
        pltpu.async_copy(dst_hbm.at[wid, pl.ds(b, 1)], didx_v.at[b], dsems[b])
        pltpu.async_copy(x_hbm.at[sidx_v.at[b]], rows_v.at[b], gsems[b])

    def _step(j, b):
        pltpu.make_async_copy(x_hbm.at[sidx_v.at[j]], rows_v.at[b],
                              gsems[b]).wait()
        pltpu.make_async_copy(dst_hbm.at[wid, pl.ds(j, 1)], didx_v.at[b],
                              dsems[b]).wait()
        pltpu.sync_copy(rows_v.at[b], acc_sh.at[didx_v.at[b, 0]], add=True)

        @pl.when(j + 2 < NCH)
        def _():
            pltpu.async_copy(dst_hbm.at[wid, pl.ds(j + 2, 1)], didx_v.at[b],
                             dsems[b])
            pltpu.async_copy(x_hbm.at[sidx_v.at[j + 2]], rows_v.at[b],
                             gsems[b])

    def _g(g, c):
        for b in range(2):
            _step(g * 2 + b, b)
        return c
    lax.fori_loop(0, NCH // 2, _g, 0)
    _step(NCH - 1, (NCH - 1) % 2)

    plsc.subcore_barrier()
    pltpu.sync_copy(acc_sh.at[pl.ds(sid * RPT, RPT), :],
                    out_hbm.at[cid, pl.ds(sid * RPT, RPT), :])


def _make_agg_body(d):
    def _agg_body(x_hbm, src_hbm, dst_hbm, zeros_hbm, out_hbm,
                  sidx_v, didx_v, rows_v, acc_sh, *sems):
        cid = lax.axis_index("c")
        sid = lax.axis_index("s")
        wid = cid * NS + sid
        pltpu.sync_copy(zeros_hbm, acc_sh.at[pl.ds(sid * RPT, RPT), :])
        pltpu.sync_copy(src_hbm.at[wid], sidx_v)
        pltpu.sync_copy(dst_hbm.at[wid], didx_v)
        plsc.subcore_barrier()

        for b in range(NBUF):
            pltpu.async_copy(x_hbm.at[sidx_v.at[b]], rows_v.at[b], sems[b])

        def _g(g, c):
            for b in range(NBUF):
                j = g * NBUF + b
                pltpu.make_async_copy(x_hbm.at[sidx_v.at[j]], rows_v.at[b],
                                      sems[b]).wait()
                pltpu.sync_copy(rows_v.at[b], acc_sh.at[didx_v.at[j]],
                                add=True)

                @pl.when(j + NBUF < NCH)
                def _():
                    pltpu.async_copy(x_hbm.at[sidx_v.at[j + NBUF]],
                                     rows_v.at[b], sems[b])
            return c
        lax.fori_loop(0, NCH // NBUF, _g, 0)

        plsc.subcore_barrier()
        pltpu.sync_copy(acc_sh.at[pl.ds(sid * RPT, RPT), :],
                        out_hbm.at[cid, pl.ds(sid * RPT, RPT), :])
    return _agg_body


def _prep_body(x_ref, ds_ref, dd_ref, xs_ref, ns_ref, nd_ref):
    ns = lax.rsqrt(jnp.maximum(ds_ref[0] + ds_ref[1], 1.0))
    nd = lax.rsqrt(jnp.maximum(dd_ref[0] + dd_ref[1], 1.0))
    xs_ref[...] = x_ref[...] * ns
    ns_ref[...] = ns
    nd_ref[...] = nd


def _mlp_body(a0_ref, nd_ref, ns_ref, w1_ref, b1_ref, w2_ref, out_ref):
    a = (a0_ref[0] + a0_ref[1]) * nd_ref[...]
    h = jnp.dot(a, w1_ref[...], preferred_element_type=jnp.float32)
    h = jnp.maximum(h + b1_ref[...], 0.0)
    out_ref[...] = jnp.dot(h * ns_ref[...], w2_ref[...], preferred_element_type=jnp.float32)


def _final_body(a_ref, nd_ref, b2_ref, out_ref):
    out_ref[...] = (a_ref[0] + a_ref[1]) * nd_ref[...] + b2_ref[...]


def kernel(x, edge_index, W1, b1, W2, b2):
    src = edge_index[0].reshape(NW, NCH, K)
    dst = edge_index[1].reshape(NW, NCH, K)
    zeros_r = jnp.zeros((RPT,), jnp.float32)
    zeros_h = jnp.zeros((RPT, D_H), jnp.float32)
    zeros_o = jnp.zeros((RPT, D_OUT), jnp.float32)

    mesh = plsc.VectorSubcoreMesh(**_MESH)
    degs_p, degd_p = pl.kernel(
        _deg_body,
        out_type=(jax.ShapeDtypeStruct((NC, NP), jnp.float32),
                  jax.ShapeDtypeStruct((NC, NP), jnp.float32)),
        mesh=mesh,
        scratch_types=(
            pltpu.VMEM((NCH, K), jnp.int32),
            pltpu.VMEM((K,), jnp.float32),
            pltpu.VMEM_SHARED((NP,), jnp.float32),
            pltpu.VMEM_SHARED((NP,), jnp.float32),
        ) + (pltpu.SemaphoreType.DMA,) * NBUF,
    )(src, dst, zeros_r)

    B = 2000
    G = N // B
    degs3 = degs_p.reshape(NC, NP, 1)
    degd3 = degd_p.reshape(NC, NP, 1)
    xs, ns, nd = pl.pallas_call(
        _prep_body,
        grid=(G,),
        in_specs=[pl.BlockSpec((B, D_IN), lambda i: (i, 0)),
                  pl.BlockSpec((NC, B, 1), lambda i: (0, i, 0)),
                  pl.BlockSpec((NC, B, 1), lambda i: (0, i, 0))],
        out_specs=[pl.BlockSpec((B, D_IN), lambda i: (i, 0)),
                   pl.BlockSpec((B, 1), lambda i: (i, 0)),
                   pl.BlockSpec((B, 1), lambda i: (i, 0))],
        out_shape=[jax.ShapeDtypeStruct((N, D_IN), jnp.float32),
                   jax.ShapeDtypeStruct((N, 1), jnp.float32),
                   jax.ShapeDtypeStruct((N, 1), jnp.float32)],
    )(x, degs3, degd3)

    mesh = plsc.VectorSubcoreMesh(**_MESH)
    agg1 = pl.kernel(
        _agg1_body,
        out_type=jax.ShapeDtypeStruct((NC, NP, D_H), jnp.float32),
        mesh=mesh,
        scratch_types=(
            pltpu.VMEM((NCH, K), jnp.int32),
            pltpu.VMEM((2, 1, K), jnp.int32),
            pltpu.VMEM((2, K, D_H), jnp.float32),
            pltpu.VMEM_SHARED((NP, D_H), jnp.float32),
        ) + (pltpu.SemaphoreType.DMA,) * 4,
    )(xs, src, dst, zeros_h)

    b1r = b1.reshape(1, D_H)
    y2 = pl.pallas_call(
        _mlp_body,
        grid=(G,),
        in_specs=[pl.BlockSpec((NC, B, D_H), lambda i: (0, i, 0)),
                  pl.BlockSpec((B, 1), lambda i: (i, 0)),
                  pl.BlockSpec((B, 1), lambda i: (i, 0)),
                  pl.BlockSpec((D_H, D_H), lambda i: (0, 0)),
                  pl.BlockSpec((1, D_H), lambda i: (0, 0)),
                  pl.BlockSpec((D_H, D_OUT), lambda i: (0, 0))],
        out_specs=pl.BlockSpec((B, D_OUT), lambda i: (i, 0)),
        out_shape=jax.ShapeDtypeStruct((N, D_OUT), jnp.float32),
    )(agg1, nd, ns, W1, b1r, W2)

    mesh = plsc.VectorSubcoreMesh(**_MESH)
    agg2 = pl.kernel(
        _make_agg_body(D_OUT),
        out_type=jax.ShapeDtypeStruct((NC, NP, D_OUT), jnp.float32),
        mesh=mesh,
        compiler_params=pltpu.CompilerParams(use_tc_tiling_on_sc=False),
        scratch_types=(
            pltpu.VMEM((NCH, K), jnp.int32),
            pltpu.VMEM((NCH, K), jnp.int32),
            pltpu.VMEM((NBUF, K, D_OUT), jnp.float32),
            pltpu.VMEM_SHARED((NP, D_OUT), jnp.float32),
        ) + (pltpu.SemaphoreType.DMA,) * NBUF,
    )(y2, src, dst, zeros_o)

    b2r = b2.reshape(1, D_OUT)
    out = pl.pallas_call(
        _final_body,
        grid=(G,),
        in_specs=[pl.BlockSpec((NC, B, D_OUT), lambda i: (0, i, 0)),
                  pl.BlockSpec((B, 1), lambda i: (i, 0)),
                  pl.BlockSpec((1, D_OUT), lambda i: (0, 0))],
        out_specs=pl.BlockSpec((B, D_OUT), lambda i: (i, 0)),
        out_shape=jax.ShapeDtypeStruct((N, D_OUT), jnp.float32),
    )(agg2, nd, b2r)
    return out


# speedup vs baseline: 15.3948x; 1.1776x over previous
import jax
import jax.numpy as jnp
from jax import lax
from jax.experimental import pallas as pl
from jax.experimental.pallas import tpu as pltpu
from jax.experimental.pallas import tpu_sc as plsc

N = 10000
E = 320000
D_IN = 128
D_H = 128
D_OUT = 16

NC = 2
NS = 16
NW = NC * NS
EPW = E // NW
K = 80
NCH = EPW // K
NP = 10240
RPT = NP // NS

_MESH = dict(core_axis_name="c", subcore_axis_name="s",
             num_cores=NC, num_subcores=NS)


NBUF = 5


def _deg_body(ei_hbm, zeros_hbm, degs_out, degd_out,
              idx_v, ones_v, degs_sh, degd_sh, *sems):
    cid = lax.axis_index("c")
    sid = lax.axis_index("s")
    wid = cid * NS + sid
    pltpu.sync_copy(zeros_hbm, degs_sh.at[pl.ds(sid * RPT, RPT)])
    pltpu.sync_copy(zeros_hbm, degd_sh.at[pl.ds(sid * RPT, RPT)])

    def _fill(i, c):
        ones_v[pl.ds(i * 16, 16)] = jnp.ones((16,), jnp.float32)
        return c
    lax.fori_loop(0, K // 16, _fill, 0)
    pltpu.sync_copy(ei_hbm.at[wid], idx_v)
    plsc.subcore_barrier()

    def _g(g, c):
        for b in range(NBUF):
            j = g * NBUF + b
            pltpu.async_copy(ones_v, degs_sh.at[idx_v.at[j, 0]], sems[b],
                             add=True)
            pltpu.async_copy(ones_v, degd_sh.at[idx_v.at[j, 1]],
                             sems[NBUF + b], add=True)
        for b in range(NBUF):
            j = g * NBUF + b
            pltpu.make_async_copy(ones_v, degs_sh.at[idx_v.at[j, 0]],
                                  sems[b]).wait()
            pltpu.make_async_copy(ones_v, degd_sh.at[idx_v.at[j, 1]],
                                  sems[NBUF + b]).wait()
        return c
    lax.fori_loop(0, NCH // NBUF, _g, 0)

    plsc.subcore_barrier()
    pltpu.sync_copy(degs_sh.at[pl.ds(sid * RPT, RPT)],
                    degs_out.at[cid, pl.ds(sid * RPT, RPT)])
    pltpu.sync_copy(degd_sh.at[pl.ds(sid * RPT, RPT)],
                    degd_out.at[cid, pl.ds(sid * RPT, RPT)])


def _agg1_body(x_hbm, ei_hbm, zeros_hbm, out_hbm,
               idx_v, rows_v, acc_sh, *sems):
    isems = sems[:5]
    gsems = sems[5:]
    cid = lax.axis_index("c")
    sid = lax.axis_index("s")
    wid = cid * NS + sid
    pltpu.sync_copy(zeros_hbm, acc_sh.at[pl.ds(sid * RPT, RPT), :])
    plsc.subcore_barrier()

    def idx_load(j, b5):
        pltpu.async_copy(ei_hbm.at[wid, pl.ds(j, 1)], idx_v.at[b5], isems[b5])

    def idx_wait(j, b5):
        pltpu.make_async_copy(ei_hbm.at[wid, pl.ds(j, 1)], idx_v.at[b5],
                              isems[b5]).wait()

    def gather(g, b5, b3):
        idx_wait(g, b5)
        pltpu.async_copy(x_hbm.at[idx_v.at[b5, 0, 0]], rows_v.at[b3],
                         gsems[b3])

    def step(j, i, do_load, do_gather):
        b3, b5 = i % 3, i % 5
        pltpu.make_async_copy(x_hbm.at[idx_v.at[b5, 0, 0]], rows_v.at[b3],
                              gsems[b3]).wait()
        pltpu.sync_copy(rows_v.at[b3], acc_sh.at[idx_v.at[b5, 0, 1]],
                        add=True)
        if do_load:
            idx_load(j + 5, b5)
        if do_gather:
            gather(j + 3, (i + 3) % 5, (i + 3) % 3)

    for b in range(5):
        idx_load(b, b)
    for c in range(3):
        gather(c, c, c)

    def _t(t, c):
        for i in range(15):
            j = t * 15 + i
            step(j, i, True, True)
        return c
    lax.fori_loop(0, NCH // 15, _t, 0)
    for j in range(NCH // 15 * 15, NCH):
        step(j, j % 15, j + 5 < NCH, j + 3 < NCH)

    plsc.subcore_barrier()
    pltpu.sync_copy(acc_sh.at[pl.ds(sid * RPT, RPT), :],
                    out_hbm.at[cid, pl.ds(sid * RPT, RPT), :])


NB2 = 10


def _agg2_body(x_hbm, ei_hbm, zeros_hbm, out_hbm,
               idx_v, rows_v, acc_sh, *sems):
    cid = lax.axis_index("c")
    sid = lax.axis_index("s")
    wid = cid * NS + sid
    pltpu.sync_copy(zeros_hbm, acc_sh.at[pl.ds(sid * RPT, RPT), :])
    pltpu.sync_copy(ei_hbm.at[wid], idx_v)
    plsc.subcore_barrier()

    for b in range(NB2):
        pltpu.async_copy(x_hbm.at[idx_v.at[b, 0]], rows_v.at[b], sems[b])

    def step(j, b, do_gather):
        pltpu.make_async_copy(x_hbm.at[idx_v.at[j, 0]], rows_v.at[b],
                              sems[b]).wait()
        pltpu.sync_copy(rows_v.at[b], acc_sh.at[idx_v.at[j, 1]], add=True)
        if do_gather:
            pltpu.async_copy(x_hbm.at[idx_v.at[j + NB2, 0]], rows_v.at[b],
                             sems[b])

    def _g(g, c):
        for b in range(NB2):
            step(g * NB2 + b, b, True)
        return c
    nfull = NCH // NB2 - 1
    lax.fori_loop(0, nfull, _g, 0)
    for j in range(nfull * NB2, NCH):
        step(j, j % NB2, j + NB2 < NCH)

    plsc.subcore_barrier()
    pltpu.sync_copy(acc_sh.at[pl.ds(sid * RPT, RPT), :],
                    out_hbm.at[cid, pl.ds(sid * RPT, RPT), :])


def _prep_body(x_ref, ds_ref, dd_ref, xs_ref, ns_ref, nd_ref):
    ns = lax.rsqrt(jnp.maximum(ds_ref[0] + ds_ref[1], 1.0))
    nd = lax.rsqrt(jnp.maximum(dd_ref[0] + dd_ref[1], 1.0))
    xs_ref[...] = x_ref[...] * ns
    ns_ref[...] = ns
    nd_ref[...] = nd


def _mlp_body(a0_ref, nd_ref, ns_ref, w1_ref, b1_ref, w2_ref, out_ref):
    a = (a0_ref[0] + a0_ref[1]) * nd_ref[...]
    h = jnp.dot(a, w1_ref[...], preferred_element_type=jnp.float32)
    h = jnp.maximum(h + b1_ref[...], 0.0)
    out_ref[...] = jnp.dot(h * ns_ref[...], w2_ref[...], preferred_element_type=jnp.float32)


def _final_body(a_ref, nd_ref, b2_ref, out_ref):
    out_ref[...] = (a_ref[0] + a_ref[1]) * nd_ref[...] + b2_ref[...]


def kernel(x, edge_index, W1, b1, W2, b2):
    ei_t = edge_index.reshape(2, NW, NCH, K).transpose(1, 2, 0, 3)
    zeros_r = jnp.zeros((RPT,), jnp.float32)
    zeros_h = jnp.zeros((RPT, D_H), jnp.float32)
    zeros_o = jnp.zeros((RPT, D_OUT), jnp.float32)

    mesh = plsc.VectorSubcoreMesh(**_MESH)
    degs_p, degd_p = pl.kernel(
        _deg_body,
        out_type=(jax.ShapeDtypeStruct((NC, NP), jnp.float32),
                  jax.ShapeDtypeStruct((NC, NP), jnp.float32)),
        mesh=mesh,
        scratch_types=(
            pltpu.VMEM((NCH, 2, K), jnp.int32),
            pltpu.VMEM((K,), jnp.float32),
            pltpu.VMEM_SHARED((NP,), jnp.float32),
            pltpu.VMEM_SHARED((NP,), jnp.float32),
        ) + (pltpu.SemaphoreType.DMA,) * (2 * NBUF),
    )(ei_t, zeros_r)

    B = 2000
    G = N // B
    degs3 = degs_p.reshape(NC, NP, 1)
    degd3 = degd_p.reshape(NC, NP, 1)
    xs, ns, nd = pl.pallas_call(
        _prep_body,
        grid=(G,),
        in_specs=[pl.BlockSpec((B, D_IN), lambda i: (i, 0)),
                  pl.BlockSpec((NC, B, 1), lambda i: (0, i, 0)),
                  pl.BlockSpec((NC, B, 1), lambda i: (0, i, 0))],
        out_specs=[pl.BlockSpec((B, D_IN), lambda i: (i, 0)),
                   pl.BlockSpec((B, 1), lambda i: (i, 0)),
                   pl.BlockSpec((B, 1), lambda i: (i, 0))],
        out_shape=[jax.ShapeDtypeStruct((N, D_IN), jnp.float32),
                   jax.ShapeDtypeStruct((N, 1), jnp.float32),
                   jax.ShapeDtypeStruct((N, 1), jnp.float32)],
    )(x, degs3, degd3)

    mesh = plsc.VectorSubcoreMesh(**_MESH)
    agg1 = pl.kernel(
        _agg1_body,
        out_type=jax.ShapeDtypeStruct((NC, NP, D_H), jnp.float32),
        mesh=mesh,
        scratch_types=(
            pltpu.VMEM((5, 1, 2, K), jnp.int32),
            pltpu.VMEM((3, K, D_H), jnp.float32),
            pltpu.VMEM_SHARED((NP, D_H), jnp.float32),
        ) + (pltpu.SemaphoreType.DMA,) * 8,
    )(xs, ei_t, zeros_h)

    b1r = b1.reshape(1, D_H)
    y2 = pl.pallas_call(
        _mlp_body,
        grid=(G,),
        in_specs=[pl.BlockSpec((NC, B, D_H), lambda i: (0, i, 0)),
                  pl.BlockSpec((B, 1), lambda i: (i, 0)),
                  pl.BlockSpec((B, 1), lambda i: (i, 0)),
                  pl.BlockSpec((D_H, D_H), lambda i: (0, 0)),
                  pl.BlockSpec((1, D_H), lambda i: (0, 0)),
                  pl.BlockSpec((D_H, D_OUT), lambda i: (0, 0))],
        out_specs=pl.BlockSpec((B, D_OUT), lambda i: (i, 0)),
        out_shape=jax.ShapeDtypeStruct((N, D_OUT), jnp.float32),
    )(agg1, nd, ns, W1, b1r, W2)

    mesh = plsc.VectorSubcoreMesh(**_MESH)
    agg2 = pl.kernel(
        _agg2_body,
        out_type=jax.ShapeDtypeStruct((NC, NP, D_OUT), jnp.float32),
        mesh=mesh,
        compiler_params=pltpu.CompilerParams(use_tc_tiling_on_sc=False),
        scratch_types=(
            pltpu.VMEM((NCH, 2, K), jnp.int32),
            pltpu.VMEM((NB2, K, D_OUT), jnp.float32),
            pltpu.VMEM_SHARED((NP, D_OUT), jnp.float32),
        ) + (pltpu.SemaphoreType.DMA,) * NB2,
    )(y2, ei_t, zeros_o)

    b2r = b2.reshape(1, D_OUT)
    out = pl.pallas_call(
        _final_body,
        grid=(G,),
        in_specs=[pl.BlockSpec((NC, B, D_OUT), lambda i: (0, i, 0)),
                  pl.BlockSpec((B, 1), lambda i: (i, 0)),
                  pl.BlockSpec((1, D_OUT), lambda i: (0, 0))],
        out_specs=pl.BlockSpec((B, D_OUT), lambda i: (i, 0)),
        out_shape=jax.ShapeDtypeStruct((N, D_OUT), jnp.float32),
    )(agg2, nd, b2r)
    return out
